# baseline (speedup 1.0000x reference)
import functools

import jax
import jax.numpy as jnp
from jax import lax
from jax.experimental import pallas as pl
from jax.experimental.pallas import tpu as pltpu
from jax.experimental.pallas import tpu_sc as plsc

_B, _S, _D = 1, 2048, 1024
_E, _TOPK = 8, 2
_FF, _FF_SH = 256, 512
_T = _B * _S

_TILE = 256
_NT = 24
_ROWS = _NT * _TILE
_NC, _NS = 2, 16
_NW = _NC * _NS
_TPW = _T // _NW


def _silu(x):
    return x * (1.0 / (1.0 + jnp.exp(-x)))


def _router_body(flat_ref, rw_ref, logits_ref, w01_ref, pos_ref, te_ref):
    flat = flat_ref[...]
    logits = jnp.dot(flat, rw_ref[...], preferred_element_type=jnp.float32)
    logits_ref[...] = logits
    lmax = jnp.max(logits, axis=1, keepdims=True)
    p = jnp.exp(logits - lmax)
    lane = lax.broadcasted_iota(jnp.int32, (_T, _E), 1)
    m1 = jnp.max(p, axis=1, keepdims=True)
    i1 = jnp.min(jnp.where(p == m1, lane, _E), axis=1, keepdims=True)
    p2 = jnp.where(lane == i1, -jnp.inf, p)
    m2 = jnp.max(p2, axis=1, keepdims=True)
    i2 = jnp.min(jnp.where(p2 == m2, lane, _E), axis=1, keepdims=True)
    s = m1 + m2
    w01_ref[...] = jnp.concatenate([m1 / s, m2 / s], axis=1)

    oh1 = (lane == i1).astype(jnp.bfloat16)
    oh2 = (lane == i2).astype(jnp.bfloat16)
    cnt = oh1 + oh2
    r_i = lax.broadcasted_iota(jnp.int32, (_T, _T), 0)
    c_i = lax.broadcasted_iota(jnp.int32, (_T, _T), 1)
    ltri = (c_i < r_i).astype(jnp.bfloat16)
    x_excl = jnp.dot(ltri, cnt, preferred_element_type=jnp.float32)

    c_tot = x_excl[_T - 1:_T, :] + cnt[_T - 1:_T, :].astype(jnp.float32)
    tiles = ((c_tot + float(_TILE - 1)) * (1.0 / _TILE)).astype(jnp.int32)
    tiles = tiles.astype(jnp.float32)
    r8 = lax.broadcasted_iota(jnp.int32, (_E, _E), 0)
    c8 = lax.broadcasted_iota(jnp.int32, (_E, _E), 1)
    diag = jnp.where(r8 == c8, jnp.broadcast_to(tiles, (_E, _E)), 0.0)
    tiles_col = jnp.sum(diag, axis=1, keepdims=True)
    start = jnp.sum(jnp.where(r8 < c8, jnp.broadcast_to(tiles_col, (_E, _E)),
                              0.0), axis=0, keepdims=True)
    aligned = start * float(_TILE)

    rank1 = jnp.sum(jnp.where(lane == i1, x_excl, 0.0), axis=1, keepdims=True)
    rank2 = jnp.sum(jnp.where(lane == i2, x_excl, 0.0), axis=1, keepdims=True)
    off1 = jnp.sum(jnp.where(lane == i1, aligned, 0.0), axis=1, keepdims=True)
    off2 = jnp.sum(jnp.where(lane == i2, aligned, 0.0), axis=1, keepdims=True)
    pos_ref[...] = jnp.concatenate(
        [rank1 + off1, rank2 + off2], axis=1).astype(jnp.int32)

    start_col = jnp.sum(jnp.where(c8 < r8, jnp.broadcast_to(tiles, (_E, _E)),
                                  0.0), axis=1, keepdims=True)
    incl_col = start_col + tiles_col
    jt = lax.broadcasted_iota(jnp.int32, (_E, _NT), 1).astype(jnp.float32)
    te = jnp.sum((jnp.broadcast_to(incl_col, (_E, _NT)) <= jt)
                 .astype(jnp.float32), axis=0, keepdims=True)
    te_ref[...] = jnp.minimum(te, float(_E - 1)).astype(jnp.int32)


def _router_tc(flat, router_w, interpret=False):
    return pl.pallas_call(
        _router_body,
        out_shape=[
            jax.ShapeDtypeStruct((_T, _E), jnp.float32),
            jax.ShapeDtypeStruct((_T, 2), jnp.float32),
            jax.ShapeDtypeStruct((_T, 2), jnp.int32),
            jax.ShapeDtypeStruct((1, _NT), jnp.int32),
        ],
        interpret=interpret,
    )(flat, router_w)


def _shared_body(flat_ref, g_ref, u_ref, d_ref, out_ref):
    flat = flat_ref[...]
    g = jnp.dot(flat, g_ref[...], preferred_element_type=jnp.float32)
    u = jnp.dot(flat, u_ref[...], preferred_element_type=jnp.float32)
    out_ref[...] = jnp.dot(_silu(g) * u, d_ref[...],
                           preferred_element_type=jnp.float32)


def _shared_tc(flat, sh_gate, sh_up, sh_down, interpret=False):
    return pl.pallas_call(
        _shared_body,
        out_shape=jax.ShapeDtypeStruct((_T, _D), jnp.float32),
        interpret=interpret,
    )(flat, sh_gate, sh_up, sh_down)


def _grouped_body(te_ref, x_ref, wg_ref, wu_ref, wd_ref, y_ref):
    x = x_ref[...]
    g = jnp.dot(x, wg_ref[0], preferred_element_type=jnp.float32)
    u = jnp.dot(x, wu_ref[0], preferred_element_type=jnp.float32)
    y_ref[...] = jnp.dot(_silu(g) * u, wd_ref[0],
                         preferred_element_type=jnp.float32)


def _grouped_tc(tile_expert, x_sorted, w_gate, w_up, w_down, interpret=False):
    grid_spec = pltpu.PrefetchScalarGridSpec(
        num_scalar_prefetch=1,
        grid=(_NT,),
        in_specs=[
            pl.BlockSpec((_TILE, _D), lambda i, te: (i, 0)),
            pl.BlockSpec((1, _D, _FF), lambda i, te: (te[i], 0, 0)),
            pl.BlockSpec((1, _D, _FF), lambda i, te: (te[i], 0, 0)),
            pl.BlockSpec((1, _FF, _D), lambda i, te: (te[i], 0, 0)),
        ],
        out_specs=pl.BlockSpec((_TILE, _D), lambda i, te: (i, 0)),
    )
    return pl.pallas_call(
        _grouped_body,
        grid_spec=grid_spec,
        out_shape=jax.ShapeDtypeStruct((_ROWS, _D), jnp.float32),
        compiler_params=pltpu.CompilerParams(
            dimension_semantics=("arbitrary",)),
        interpret=interpret,
    )(tile_expert, x_sorted, w_gate, w_up, w_down)


def _dispatch_sc(flat, pos3):
    mesh = plsc.VectorSubcoreMesh(core_axis_name="c", subcore_axis_name="s")

    @functools.partial(
        pl.kernel, mesh=mesh,
        out_type=jax.ShapeDtypeStruct((_ROWS, _D), jnp.float32),
        scratch_types=[
            pltpu.VMEM((2, _TPW), jnp.int32),
            pltpu.VMEM((_TPW, _D), jnp.float32),
            pltpu.SemaphoreType.DMA,
        ],
    )
    def k(flat_hbm, pos_hbm, out_hbm, idx_v, rows_v, sem):
        wid = lax.axis_index("s") * _NC + lax.axis_index("c")
        base = wid * _TPW
        pltpu.sync_copy(pos_hbm.at[wid], idx_v)
        pltpu.sync_copy(flat_hbm.at[pl.ds(base, _TPW)], rows_v)
        pltpu.async_copy(rows_v, out_hbm.at[idx_v.at[0]], sem).wait()
        pltpu.async_copy(rows_v, out_hbm.at[idx_v.at[1]], sem).wait()

    return k(flat, pos3)


def _gather_sc(y, pos3):
    mesh = plsc.VectorSubcoreMesh(core_axis_name="c", subcore_axis_name="s")

    @functools.partial(
        pl.kernel, mesh=mesh,
        out_type=[jax.ShapeDtypeStruct((_T, _D), jnp.float32),
                  jax.ShapeDtypeStruct((_T, _D), jnp.float32)],
        scratch_types=[
            pltpu.VMEM((2, _TPW), jnp.int32),
            pltpu.VMEM((_TPW, _D), jnp.float32),
            pltpu.SemaphoreType.DMA,
        ],
    )
    def k(y_hbm, pos_hbm, y0_hbm, y1_hbm, idx_v, rows_v, sem):
        wid = lax.axis_index("s") * _NC + lax.axis_index("c")
        base = wid * _TPW
        pltpu.sync_copy(pos_hbm.at[wid], idx_v)
        pltpu.async_copy(y_hbm.at[idx_v.at[0]], rows_v, sem).wait()
        pltpu.sync_copy(rows_v, y0_hbm.at[pl.ds(base, _TPW)])
        pltpu.async_copy(y_hbm.at[idx_v.at[1]], rows_v, sem).wait()
        pltpu.sync_copy(rows_v, y1_hbm.at[pl.ds(base, _TPW)])

    return k(y, pos3)


def _combine_body(sh_ref, y0_ref, y1_ref, w01_ref, out_ref):
    w0 = w01_ref[:, 0:1]
    w1 = w01_ref[:, 1:2]
    out_ref[...] = sh_ref[...] + w0 * y0_ref[...] + w1 * y1_ref[...]


def _combine_tc(shared, y0, y1, w01, interpret=False):
    nblk = 4
    rows = _T // nblk
    return pl.pallas_call(
        _combine_body,
        grid=(nblk,),
        in_specs=[
            pl.BlockSpec((rows, _D), lambda i: (i, 0)),
            pl.BlockSpec((rows, _D), lambda i: (i, 0)),
            pl.BlockSpec((rows, _D), lambda i: (i, 0)),
            pl.BlockSpec((rows, 2), lambda i: (i, 0)),
        ],
        out_specs=pl.BlockSpec((rows, _D), lambda i: (i, 0)),
        out_shape=jax.ShapeDtypeStruct((_T, _D), jnp.float32),
        interpret=interpret,
    )(shared, y0, y1, w01)


@jax.jit
def kernel(hidden_states, router_w, w_gate, w_up, w_down,
           sh_gate, sh_up, sh_down):
    flat = hidden_states.reshape(_T, _D)
    logits, w01, pos01, te = _router_tc(flat, router_w)
    pos3 = pos01.reshape(_NW, _TPW, 2).transpose(0, 2, 1)
    tile_expert = te.reshape(_NT)
    x_sorted = _dispatch_sc(flat, pos3)
    shared = _shared_tc(flat, sh_gate, sh_up, sh_down)
    y = _grouped_tc(tile_expert, x_sorted, w_gate, w_up, w_down)
    y0, y1 = _gather_sc(y, pos3)
    out = _combine_tc(shared, y0, y1, w01)
    return out.reshape(_B, _S, _D), logits
